# Initial kernel scaffold; baseline (speedup 1.0000x reference)
import jax
import jax.numpy as jnp
from jax.experimental import pallas as pl


def kernel(dynamic_features, static_features, edge_index, W_ih0, W_hh0, b_ih0, b_hh0, W_ih1, W_hh1, b_ih1, b_hh1, Ws, bs, Wc, bc, Wsp, bsp, W1, b1, W2, b2, Wl, bl):
    raise NotImplementedError("write your pallas kernel here")


# speedup vs baseline: 5.5793x; 5.5793x over previous
import functools

import numpy as np

import jax
import jax.numpy as jnp
from jax import lax
from jax.experimental import pallas as pl
from jax.experimental.pallas import tpu as pltpu
from jax.experimental.pallas import tpu_sc as plsc

F32 = jnp.float32
_I0 = np.int32(0)
B, N, T, DD, DS, HL, HG, E = 2, 10000, 24, 16, 32, 128, 128, 160000
EPAD = 163840
NDUM = 10240
R1 = 400
R2 = 1000
CHUNK = 128


def _enc_body(x_ref, xs_ref, wih0_r, whh0_r, b0_r, wih1_r, whh1_r, b1_r,
              ws_r, bs_r, wch_r, wcs_r, bc_r, out_ref):
    wih0 = wih0_r[...]
    whh0 = whh0_r[...]
    wih1 = wih1_r[...]
    whh1 = whh1_r[...]
    b0 = b0_r[...]
    b1 = b1_r[...]
    h0 = jnp.zeros((R1, HL), F32)
    c0 = jnp.zeros((R1, HL), F32)
    h1 = jnp.zeros((R1, HL), F32)
    c1 = jnp.zeros((R1, HL), F32)
    for t in range(T):
        xt = x_ref[:, t * DD:(t + 1) * DD]
        g = (jnp.dot(xt, wih0, preferred_element_type=F32)
             + jnp.dot(h0, whh0, preferred_element_type=F32) + b0)
        ig = jax.nn.sigmoid(g[:, 0:HL])
        fg = jax.nn.sigmoid(g[:, HL:2 * HL])
        gg = jnp.tanh(g[:, 2 * HL:3 * HL])
        og = jax.nn.sigmoid(g[:, 3 * HL:4 * HL])
        c0 = fg * c0 + ig * gg
        h0 = og * jnp.tanh(c0)
        g = (jnp.dot(h0, wih1, preferred_element_type=F32)
             + jnp.dot(h1, whh1, preferred_element_type=F32) + b1)
        ig = jax.nn.sigmoid(g[:, 0:HL])
        fg = jax.nn.sigmoid(g[:, HL:2 * HL])
        gg = jnp.tanh(g[:, 2 * HL:3 * HL])
        og = jax.nn.sigmoid(g[:, 3 * HL:4 * HL])
        c1 = fg * c1 + ig * gg
        h1 = og * jnp.tanh(c1)
    s = jnp.maximum(jnp.dot(xs_ref[...], ws_r[...],
                            preferred_element_type=F32) + bs_r[...], 0.0)
    comb = jnp.maximum(jnp.dot(h1, wch_r[...], preferred_element_type=F32)
                       + jnp.dot(s, wcs_r[...], preferred_element_type=F32)
                       + bc_r[...], 0.0)
    out_ref[...] = comb


def _encode(x2d, xs, wih0T, whh0T, b0, wih1T, whh1T, b1s, wsT, bs2,
            wchT, wcsT, bc2):
    full = lambda shp: pl.BlockSpec(shp, lambda i: tuple(_I0 for _ in shp))
    return pl.pallas_call(
        _enc_body,
        grid=(B * N // R1,),
        in_specs=[
            pl.BlockSpec((R1, T * DD), lambda i: (i, _I0)),
            pl.BlockSpec((R1, DS), lambda i: (i, _I0)),
            full((DD, 4 * HL)), full((HL, 4 * HL)), full((1, 4 * HL)),
            full((HL, 4 * HL)), full((HL, 4 * HL)), full((1, 4 * HL)),
            full((DS, HL // 2)), full((1, HL // 2)),
            full((HL, HL)), full((HL // 2, HL)), full((1, HL)),
        ],
        out_specs=pl.BlockSpec((R1, HL), lambda i: (i, _I0)),
        out_shape=jax.ShapeDtypeStruct((B * N, HL), F32),
    )(x2d, xs, wih0T, whh0T, b0, wih1T, whh1T, b1s, wsT, bs2, wchT, wcsT, bc2)


def _pre_body(deg_r, comb_r, sg_r, wsp_r, bsp_r, w1h_r, w1p_r,
              h1_ref, dinv_ref):
    d = deg_r[0][:, 0:1] + 1.0
    dinv = lax.rsqrt(d)
    dinv_ref[...] = dinv
    ps = jnp.maximum(jnp.dot(sg_r[...], wsp_r[...],
                             preferred_element_type=F32) + bsp_r[...], 0.0)
    pw = jnp.dot(ps, w1p_r[...], preferred_element_type=F32)
    for b in range(B):
        h = jnp.dot(comb_r[b], w1h_r[...], preferred_element_type=F32) + pw
        h1_ref[b] = dinv * h


def _gcn_pre(deg2, comb3, sg, wspT, bsp2, w1hT, w1pT):
    full = lambda shp: pl.BlockSpec(shp, lambda i: tuple(_I0 for _ in shp))
    return pl.pallas_call(
        _pre_body,
        grid=(N // R2,),
        in_specs=[
            pl.BlockSpec((B, R2, HG), lambda i: (_I0, i, _I0)),
            pl.BlockSpec((B, R2, HL), lambda i: (_I0, i, _I0)),
            pl.BlockSpec((R2, DS), lambda i: (i, _I0)),
            full((DS, DS // 4)), full((1, DS // 4)),
            full((HL, HG)), full((DS // 4, HG)),
        ],
        out_specs=[
            pl.BlockSpec((B, R2, HG), lambda i: (_I0, i, _I0)),
            pl.BlockSpec((R2, 1), lambda i: (i, _I0)),
        ],
        out_shape=[
            jax.ShapeDtypeStruct((B, N, HG), F32),
            jax.ShapeDtypeStruct((N, 1), F32),
        ],
    )(deg2, comb3, sg, wspT, bsp2, w1hT, w1pT)


def _mid_body(agg_r, h_r, dinv_r, b_r, w2_r, out_ref):
    dinv = dinv_r[...]
    for b in range(B):
        x = jnp.maximum(dinv * (agg_r[b] + h_r[b]) + b_r[...], 0.0)
        out_ref[b] = dinv * jnp.dot(x, w2_r[...], preferred_element_type=F32)


def _gcn_mid(agg1, h1s, dinv, b1v, w2T):
    full = lambda shp: pl.BlockSpec(shp, lambda i: tuple(_I0 for _ in shp))
    return pl.pallas_call(
        _mid_body,
        grid=(N // R2,),
        in_specs=[
            pl.BlockSpec((B, R2, HG), lambda i: (_I0, i, _I0)),
            pl.BlockSpec((B, R2, HG), lambda i: (_I0, i, _I0)),
            pl.BlockSpec((R2, 1), lambda i: (i, _I0)),
            full((1, HG)), full((HG, HG)),
        ],
        out_specs=pl.BlockSpec((B, R2, HG), lambda i: (_I0, i, _I0)),
        out_shape=jax.ShapeDtypeStruct((B, N, HG), F32),
    )(agg1, h1s, dinv, b1v, w2T)


def _fin_body(agg_r, h_r, dinv_r, b_r, wl_r, bl_r, out_ref):
    dinv = dinv_r[...]
    for b in range(B):
        x = jnp.maximum(dinv * (agg_r[b] + h_r[b]) + b_r[...], 0.0)
        out_ref[b] = jnp.dot(x, wl_r[...], preferred_element_type=F32) + bl_r[...]


def _gcn_fin(agg2, h2s, dinv, b2v, wlT, blv):
    full = lambda shp: pl.BlockSpec(shp, lambda i: tuple(_I0 for _ in shp))
    return pl.pallas_call(
        _fin_body,
        grid=(N // R2,),
        in_specs=[
            pl.BlockSpec((B, R2, HG), lambda i: (_I0, i, _I0)),
            pl.BlockSpec((B, R2, HG), lambda i: (_I0, i, _I0)),
            pl.BlockSpec((R2, 1), lambda i: (i, _I0)),
            full((1, HG)), full((HG, 1)), full((1, 1)),
        ],
        out_specs=pl.BlockSpec((B, R2, 1), lambda i: (_I0, i, _I0)),
        out_shape=jax.ShapeDtypeStruct((B, N, 1), F32),
    )(agg2, h2s, dinv, b2v, wlT, blv)


_EPS = EPAD // 16
_EPW = EPAD // 32
_ZR = NDUM // 16


def _make_agg_body(width):
    def _agg_body(h_hbm, srcs_hbm, dst_hbm, zeros_hbm, out_hbm,
                  src_v, dst_v, rows_v, acc_s, sem):
        c = lax.axis_index("c")
        s = lax.axis_index("s")
        pltpu.sync_copy(zeros_hbm.at[pl.ds(s * _ZR, _ZR)],
                        acc_s.at[pl.ds(s * _ZR, _ZR)])
        plsc.subcore_barrier()
        sbase = c * EPAD + s * _EPS
        dbase = s * _EPS
        def body(i, carry):
            pltpu.sync_copy(srcs_hbm.at[pl.ds(sbase + i * CHUNK, CHUNK)],
                            src_v)
            pltpu.sync_copy(dst_hbm.at[pl.ds(dbase + i * CHUNK, CHUNK)],
                            dst_v)
            pltpu.async_copy(h_hbm.at[src_v], rows_v, sem).wait()
            pltpu.sync_copy(rows_v, acc_s.at[dst_v], add=True)
            return carry
        lax.fori_loop(jnp.int32(0), jnp.int32(_EPS // CHUNK), body,
                      jnp.int32(0))
        plsc.subcore_barrier()
        pltpu.sync_copy(acc_s.at[pl.ds(s * _ZR, _ZR)],
                        out_hbm.at[c, pl.ds(s * _ZR, _ZR)])
    return _agg_body


@functools.lru_cache(maxsize=None)
def _build_agg_kernel(width):
    mesh = plsc.VectorSubcoreMesh(core_axis_name="c", subcore_axis_name="s")
    return functools.partial(
        pl.kernel,
        mesh=mesh,
        out_type=jax.ShapeDtypeStruct((B, NDUM, width), F32),
        scratch_types=[
            pltpu.VMEM((CHUNK,), jnp.int32),
            pltpu.VMEM((CHUNK,), jnp.int32),
            pltpu.VMEM((CHUNK, width), F32),
            pltpu.VMEM_SHARED((NDUM, width), F32),
            pltpu.SemaphoreType.DMA,
        ],
    )(_make_agg_body(width))


def _deg_call(ones_flat, srcs2, dstp, zeros128):
    return _build_agg_kernel(HG)(ones_flat, srcs2, dstp, zeros128)[:, :N]


def _agg_call(hflat, srcs2, dstp, zeros128):
    return _build_agg_kernel(HG)(hflat, srcs2, dstp, zeros128)[:, :N]


def kernel(dynamic_features, static_features, edge_index, W_ih0, W_hh0, b_ih0,
           b_hh0, W_ih1, W_hh1, b_ih1, b_hh1, Ws, bs, Wc, bc, Wsp, bsp,
           W1, b1, W2, b2, Wl, bl):
    x2d = dynamic_features.reshape(B * N, T * DD)
    xs = static_features.reshape(B * N, DS)
    sg = static_features[0]

    src = edge_index[0].astype(jnp.int32)
    dst = edge_index[1].astype(jnp.int32)
    pad = EPAD - E
    srcp = jnp.concatenate([src, jnp.zeros((pad,), jnp.int32)])
    dstp = jnp.concatenate([dst, jnp.full((pad,), N, jnp.int32)])
    srcs2 = jnp.concatenate([srcp, srcp + N])
    zeros128 = jnp.zeros((NDUM, HG), F32)
    ones_flat = jnp.ones((B * N, HG), F32)

    b0 = (b_ih0 + b_hh0).reshape(1, 4 * HL)
    b1s = (b_ih1 + b_hh1).reshape(1, 4 * HL)
    comb = _encode(x2d, xs, W_ih0.T, W_hh0.T, b0, W_ih1.T, W_hh1.T, b1s,
                   Ws.T, bs.reshape(1, -1), Wc[:, :HL].T, Wc[:, HL:].T,
                   bc.reshape(1, -1))

    deg2 = _deg_call(ones_flat, srcs2, dstp, zeros128)

    h1s, dinv = _gcn_pre(deg2, comb.reshape(B, N, HL), sg, Wsp.T,
                         bsp.reshape(1, -1), W1[:, :HL].T, W1[:, HL:].T)

    agg1 = _agg_call(h1s.reshape(B * N, HG), srcs2, dstp, zeros128)

    h2s = _gcn_mid(agg1, h1s, dinv, b1.reshape(1, -1), W2.T)

    agg2 = _agg_call(h2s.reshape(B * N, HG), srcs2, dstp, zeros128)

    res = _gcn_fin(agg2, h2s, dinv, b2.reshape(1, -1), Wl.T, bl.reshape(1, 1))
    return res[:, :, 0]


# speedup vs baseline: 6.2487x; 1.1200x over previous
import functools

import numpy as np

import jax
import jax.numpy as jnp
from jax import lax
from jax.experimental import pallas as pl
from jax.experimental.pallas import tpu as pltpu
from jax.experimental.pallas import tpu_sc as plsc

F32 = jnp.float32
_I0 = np.int32(0)
B, N, T, DD, DS, HL, HG, E = 2, 10000, 24, 16, 32, 128, 128, 160000
EPAD = 163840
NDUM = 10240
R1 = 400
R2 = 1000
CHUNK = 128


def _enc_body(x_ref, xs_ref, wih0_r, whh0_r, b0_r, wih1_r, whh1_r, b1_r,
              ws_r, bs_r, wch_r, wcs_r, bc_r, out_ref):
    wih0 = wih0_r[...]
    whh0 = whh0_r[...]
    wih1 = wih1_r[...]
    whh1 = whh1_r[...]
    b0 = b0_r[...]
    b1 = b1_r[...]
    h0 = jnp.zeros((R1, HL), F32)
    c0 = jnp.zeros((R1, HL), F32)
    h1 = jnp.zeros((R1, HL), F32)
    c1 = jnp.zeros((R1, HL), F32)
    for t in range(T):
        xt = x_ref[:, t * DD:(t + 1) * DD]
        g = (jnp.dot(xt, wih0, preferred_element_type=F32)
             + jnp.dot(h0, whh0, preferred_element_type=F32) + b0)
        ig = jax.nn.sigmoid(g[:, 0:HL])
        fg = jax.nn.sigmoid(g[:, HL:2 * HL])
        gg = jnp.tanh(g[:, 2 * HL:3 * HL])
        og = jax.nn.sigmoid(g[:, 3 * HL:4 * HL])
        c0 = fg * c0 + ig * gg
        h0 = og * jnp.tanh(c0)
        g = (jnp.dot(h0, wih1, preferred_element_type=F32)
             + jnp.dot(h1, whh1, preferred_element_type=F32) + b1)
        ig = jax.nn.sigmoid(g[:, 0:HL])
        fg = jax.nn.sigmoid(g[:, HL:2 * HL])
        gg = jnp.tanh(g[:, 2 * HL:3 * HL])
        og = jax.nn.sigmoid(g[:, 3 * HL:4 * HL])
        c1 = fg * c1 + ig * gg
        h1 = og * jnp.tanh(c1)
    s = jnp.maximum(jnp.dot(xs_ref[...], ws_r[...],
                            preferred_element_type=F32) + bs_r[...], 0.0)
    comb = jnp.maximum(jnp.dot(h1, wch_r[...], preferred_element_type=F32)
                       + jnp.dot(s, wcs_r[...], preferred_element_type=F32)
                       + bc_r[...], 0.0)
    out_ref[...] = comb


def _encode(x2d, xs, wih0T, whh0T, b0, wih1T, whh1T, b1s, wsT, bs2,
            wchT, wcsT, bc2):
    full = lambda shp: pl.BlockSpec(shp, lambda i: tuple(_I0 for _ in shp))
    return pl.pallas_call(
        _enc_body,
        grid=(B * N // R1,),
        in_specs=[
            pl.BlockSpec((R1, T * DD), lambda i: (i, _I0)),
            pl.BlockSpec((R1, DS), lambda i: (i, _I0)),
            full((DD, 4 * HL)), full((HL, 4 * HL)), full((1, 4 * HL)),
            full((HL, 4 * HL)), full((HL, 4 * HL)), full((1, 4 * HL)),
            full((DS, HL // 2)), full((1, HL // 2)),
            full((HL, HL)), full((HL // 2, HL)), full((1, HL)),
        ],
        out_specs=pl.BlockSpec((R1, HL), lambda i: (i, _I0)),
        out_shape=jax.ShapeDtypeStruct((B * N, HL), F32),
    )(x2d, xs, wih0T, whh0T, b0, wih1T, whh1T, b1s, wsT, bs2, wchT, wcsT, bc2)


def _pre_body(deg_r, comb_r, sg_r, wsp_r, bsp_r, w1h_r, w1p_r,
              h1_ref, dinv_ref):
    d = deg_r[0][:, 0:1] + 1.0
    dinv = lax.rsqrt(d)
    dinv_ref[...] = dinv
    ps = jnp.maximum(jnp.dot(sg_r[...], wsp_r[...],
                             preferred_element_type=F32) + bsp_r[...], 0.0)
    pw = jnp.dot(ps, w1p_r[...], preferred_element_type=F32)
    for b in range(B):
        h = jnp.dot(comb_r[b], w1h_r[...], preferred_element_type=F32) + pw
        h1_ref[b] = dinv * h


def _gcn_pre(deg2, comb3, sg, wspT, bsp2, w1hT, w1pT):
    full = lambda shp: pl.BlockSpec(shp, lambda i: tuple(_I0 for _ in shp))
    return pl.pallas_call(
        _pre_body,
        grid=(N // R2,),
        in_specs=[
            pl.BlockSpec((B, R2, HG), lambda i: (_I0, i, _I0)),
            pl.BlockSpec((B, R2, HL), lambda i: (_I0, i, _I0)),
            pl.BlockSpec((R2, DS), lambda i: (i, _I0)),
            full((DS, DS // 4)), full((1, DS // 4)),
            full((HL, HG)), full((DS // 4, HG)),
        ],
        out_specs=[
            pl.BlockSpec((B, R2, HG), lambda i: (_I0, i, _I0)),
            pl.BlockSpec((R2, 1), lambda i: (i, _I0)),
        ],
        out_shape=[
            jax.ShapeDtypeStruct((B, N, HG), F32),
            jax.ShapeDtypeStruct((N, 1), F32),
        ],
    )(deg2, comb3, sg, wspT, bsp2, w1hT, w1pT)


def _mid_body(agg_r, h_r, dinv_r, b_r, w2_r, out_ref):
    dinv = dinv_r[...]
    for b in range(B):
        x = jnp.maximum(dinv * (agg_r[b] + h_r[b]) + b_r[...], 0.0)
        out_ref[b] = dinv * jnp.dot(x, w2_r[...], preferred_element_type=F32)


def _gcn_mid(agg1, h1s, dinv, b1v, w2T):
    full = lambda shp: pl.BlockSpec(shp, lambda i: tuple(_I0 for _ in shp))
    return pl.pallas_call(
        _mid_body,
        grid=(N // R2,),
        in_specs=[
            pl.BlockSpec((B, R2, HG), lambda i: (_I0, i, _I0)),
            pl.BlockSpec((B, R2, HG), lambda i: (_I0, i, _I0)),
            pl.BlockSpec((R2, 1), lambda i: (i, _I0)),
            full((1, HG)), full((HG, HG)),
        ],
        out_specs=pl.BlockSpec((B, R2, HG), lambda i: (_I0, i, _I0)),
        out_shape=jax.ShapeDtypeStruct((B, N, HG), F32),
    )(agg1, h1s, dinv, b1v, w2T)


def _fin_body(agg_r, h_r, dinv_r, b_r, wl_r, bl_r, out_ref):
    dinv = dinv_r[...]
    for b in range(B):
        x = jnp.maximum(dinv * (agg_r[b] + h_r[b]) + b_r[...], 0.0)
        out_ref[b] = jnp.dot(x, wl_r[...], preferred_element_type=F32) + bl_r[...]


def _gcn_fin(agg2, h2s, dinv, b2v, wlT, blv):
    full = lambda shp: pl.BlockSpec(shp, lambda i: tuple(_I0 for _ in shp))
    return pl.pallas_call(
        _fin_body,
        grid=(N // R2,),
        in_specs=[
            pl.BlockSpec((B, R2, HG), lambda i: (_I0, i, _I0)),
            pl.BlockSpec((B, R2, HG), lambda i: (_I0, i, _I0)),
            pl.BlockSpec((R2, 1), lambda i: (i, _I0)),
            full((1, HG)), full((HG, 1)), full((1, 1)),
        ],
        out_specs=pl.BlockSpec((B, R2, 1), lambda i: (_I0, i, _I0)),
        out_shape=jax.ShapeDtypeStruct((B, N, 1), F32),
    )(agg2, h2s, dinv, b2v, wlT, blv)


_EPS = EPAD // 16
_EPW = EPAD // 32
_ZR = NDUM // 16


_NCH = _EPS // CHUNK
_GK = 2


def _make_agg_body(width):
    def _agg_body(h_hbm, srcs_hbm, dst_hbm, zeros_hbm, out_hbm,
                  src_v, dstb, bufs, acc_s, gsem, ssem):
        c = lax.axis_index("c")
        s = lax.axis_index("s")
        pltpu.sync_copy(zeros_hbm.at[pl.ds(s * _ZR, _ZR)],
                        acc_s.at[pl.ds(s * _ZR, _ZR)])
        pltpu.sync_copy(srcs_hbm.at[c, pl.ds(s * _NCH, _NCH)], src_v)
        dbase = s * _EPS
        plsc.subcore_barrier()

        @pl.loop(jnp.int32(0), jnp.int32(_NCH), step=jnp.int32(_GK))
        def _group(i0):
            for j in range(_GK):
                pltpu.make_async_copy(
                    dst_hbm.at[pl.ds(dbase + (i0 + np.int32(j)) * CHUNK,
                                     CHUNK)],
                    dstb.at[np.int32(j)], gsem).start()
                pltpu.make_async_copy(h_hbm.at[src_v.at[i0 + np.int32(j)]],
                                      bufs.at[np.int32(j)], gsem).start()
            for j in range(_GK):
                pltpu.make_async_copy(
                    dst_hbm.at[pl.ds(dbase + (i0 + np.int32(j)) * CHUNK,
                                     CHUNK)],
                    dstb.at[np.int32(j)], gsem).wait()
                pltpu.make_async_copy(h_hbm.at[src_v.at[i0 + np.int32(j)]],
                                      bufs.at[np.int32(j)], gsem).wait()
            for j in range(_GK):
                pltpu.make_async_copy(bufs.at[np.int32(j)],
                                      acc_s.at[dstb.at[np.int32(j)]],
                                      ssem).start(add=True)
            for j in range(_GK):
                pltpu.make_async_copy(bufs.at[np.int32(j)],
                                      acc_s.at[dstb.at[np.int32(j)]],
                                      ssem).wait()

        plsc.subcore_barrier()
        pltpu.sync_copy(acc_s.at[pl.ds(s * _ZR, _ZR)],
                        out_hbm.at[c, pl.ds(s * _ZR, _ZR)])
    return _agg_body


@functools.lru_cache(maxsize=None)
def _build_agg_kernel(width):
    mesh = plsc.VectorSubcoreMesh(core_axis_name="c", subcore_axis_name="s")
    return functools.partial(
        pl.kernel,
        mesh=mesh,
        out_type=jax.ShapeDtypeStruct((B, NDUM, width), F32),
        scratch_types=[
            pltpu.VMEM((_NCH, CHUNK), jnp.int32),
            pltpu.VMEM((_GK, CHUNK), jnp.int32),
            pltpu.VMEM((_GK, CHUNK, width), F32),
            pltpu.VMEM_SHARED((NDUM, width), F32),
            pltpu.SemaphoreType.DMA,
            pltpu.SemaphoreType.DMA,
        ],
    )(_make_agg_body(width))


def _deg_call(ones_flat, srcs2, dstp, zeros128):
    return _build_agg_kernel(HG)(ones_flat, srcs2, dstp, zeros128)[:, :N]


def _agg_call(hflat, srcs2, dstp, zeros128):
    return _build_agg_kernel(HG)(hflat, srcs2, dstp, zeros128)[:, :N]


def kernel(dynamic_features, static_features, edge_index, W_ih0, W_hh0, b_ih0,
           b_hh0, W_ih1, W_hh1, b_ih1, b_hh1, Ws, bs, Wc, bc, Wsp, bsp,
           W1, b1, W2, b2, Wl, bl):
    x2d = dynamic_features.reshape(B * N, T * DD)
    xs = static_features.reshape(B * N, DS)
    sg = static_features[0]

    src = edge_index[0].astype(jnp.int32)
    dst = edge_index[1].astype(jnp.int32)
    pad = EPAD - E
    srcp = jnp.concatenate([src, jnp.zeros((pad,), jnp.int32)])
    dstp = jnp.concatenate([dst, jnp.full((pad,), N, jnp.int32)])
    srcs2 = jnp.stack([srcp, srcp + N]).reshape(B, EPAD // CHUNK, CHUNK)
    zeros128 = jnp.zeros((NDUM, HG), F32)
    ones_flat = jnp.ones((B * N, HG), F32)

    b0 = (b_ih0 + b_hh0).reshape(1, 4 * HL)
    b1s = (b_ih1 + b_hh1).reshape(1, 4 * HL)
    comb = _encode(x2d, xs, W_ih0.T, W_hh0.T, b0, W_ih1.T, W_hh1.T, b1s,
                   Ws.T, bs.reshape(1, -1), Wc[:, :HL].T, Wc[:, HL:].T,
                   bc.reshape(1, -1))

    deg2 = _deg_call(ones_flat, srcs2, dstp, zeros128)

    h1s, dinv = _gcn_pre(deg2, comb.reshape(B, N, HL), sg, Wsp.T,
                         bsp.reshape(1, -1), W1[:, :HL].T, W1[:, HL:].T)

    agg1 = _agg_call(h1s.reshape(B * N, HG), srcs2, dstp, zeros128)

    h2s = _gcn_mid(agg1, h1s, dinv, b1.reshape(1, -1), W2.T)

    agg2 = _agg_call(h2s.reshape(B * N, HG), srcs2, dstp, zeros128)

    res = _gcn_fin(agg2, h2s, dinv, b2.reshape(1, -1), Wl.T, bl.reshape(1, 1))
    return res[:, :, 0]


# speedup vs baseline: 6.4068x; 1.0253x over previous
import functools

import numpy as np

import jax
import jax.numpy as jnp
from jax import lax
from jax.experimental import pallas as pl
from jax.experimental.pallas import tpu as pltpu
from jax.experimental.pallas import tpu_sc as plsc

F32 = jnp.float32
_I0 = np.int32(0)
B, N, T, DD, DS, HL, HG, E = 2, 10000, 24, 16, 32, 128, 128, 160000
EPAD = 163840
NDUM = 10240
R1 = 400
R2 = 1000
CHUNK = 128


def _enc_body(x_ref, xs_ref, wih0_r, whh0_r, b0_r, wih1_r, whh1_r, b1_r,
              ws_r, bs_r, wch_r, wcs_r, bc_r, out_ref):
    wih0 = wih0_r[...]
    whh0 = whh0_r[...]
    wih1 = wih1_r[...]
    whh1 = whh1_r[...]
    b0 = b0_r[...]
    b1 = b1_r[...]
    h0 = jnp.zeros((R1, HL), F32)
    c0 = jnp.zeros((R1, HL), F32)
    h1 = jnp.zeros((R1, HL), F32)
    c1 = jnp.zeros((R1, HL), F32)
    for t in range(T):
        xt = x_ref[:, t * DD:(t + 1) * DD]
        g = (jnp.dot(xt, wih0, preferred_element_type=F32)
             + jnp.dot(h0, whh0, preferred_element_type=F32) + b0)
        ig = jax.nn.sigmoid(g[:, 0:HL])
        fg = jax.nn.sigmoid(g[:, HL:2 * HL])
        gg = jnp.tanh(g[:, 2 * HL:3 * HL])
        og = jax.nn.sigmoid(g[:, 3 * HL:4 * HL])
        c0 = fg * c0 + ig * gg
        h0 = og * jnp.tanh(c0)
        g = (jnp.dot(h0, wih1, preferred_element_type=F32)
             + jnp.dot(h1, whh1, preferred_element_type=F32) + b1)
        ig = jax.nn.sigmoid(g[:, 0:HL])
        fg = jax.nn.sigmoid(g[:, HL:2 * HL])
        gg = jnp.tanh(g[:, 2 * HL:3 * HL])
        og = jax.nn.sigmoid(g[:, 3 * HL:4 * HL])
        c1 = fg * c1 + ig * gg
        h1 = og * jnp.tanh(c1)
    s = jnp.maximum(jnp.dot(xs_ref[...], ws_r[...],
                            preferred_element_type=F32) + bs_r[...], 0.0)
    comb = jnp.maximum(jnp.dot(h1, wch_r[...], preferred_element_type=F32)
                       + jnp.dot(s, wcs_r[...], preferred_element_type=F32)
                       + bc_r[...], 0.0)
    out_ref[...] = comb


def _encode(x2d, xs, wih0T, whh0T, b0, wih1T, whh1T, b1s, wsT, bs2,
            wchT, wcsT, bc2):
    full = lambda shp: pl.BlockSpec(shp, lambda i: tuple(_I0 for _ in shp))
    return pl.pallas_call(
        _enc_body,
        grid=(B * N // R1,),
        in_specs=[
            pl.BlockSpec((R1, T * DD), lambda i: (i, _I0)),
            pl.BlockSpec((R1, DS), lambda i: (i, _I0)),
            full((DD, 4 * HL)), full((HL, 4 * HL)), full((1, 4 * HL)),
            full((HL, 4 * HL)), full((HL, 4 * HL)), full((1, 4 * HL)),
            full((DS, HL // 2)), full((1, HL // 2)),
            full((HL, HL)), full((HL // 2, HL)), full((1, HL)),
        ],
        out_specs=pl.BlockSpec((R1, HL), lambda i: (i, _I0)),
        out_shape=jax.ShapeDtypeStruct((B * N, HL), F32),
    )(x2d, xs, wih0T, whh0T, b0, wih1T, whh1T, b1s, wsT, bs2, wchT, wcsT, bc2)


def _pre_body(deg_r, comb_r, sg_r, wsp_r, bsp_r, w1h_r, w1p_r,
              h1_ref, dinv_ref):
    d = deg_r[0][:, 0:1] + 1.0
    dinv = lax.rsqrt(d)
    dinv_ref[...] = dinv
    ps = jnp.maximum(jnp.dot(sg_r[...], wsp_r[...],
                             preferred_element_type=F32) + bsp_r[...], 0.0)
    pw = jnp.dot(ps, w1p_r[...], preferred_element_type=F32)
    for b in range(B):
        h = jnp.dot(comb_r[b], w1h_r[...], preferred_element_type=F32) + pw
        h1_ref[b] = dinv * h


def _gcn_pre(deg2, comb3, sg, wspT, bsp2, w1hT, w1pT):
    full = lambda shp: pl.BlockSpec(shp, lambda i: tuple(_I0 for _ in shp))
    return pl.pallas_call(
        _pre_body,
        grid=(N // R2,),
        in_specs=[
            pl.BlockSpec((B, R2, HG), lambda i: (_I0, i, _I0)),
            pl.BlockSpec((B, R2, HL), lambda i: (_I0, i, _I0)),
            pl.BlockSpec((R2, DS), lambda i: (i, _I0)),
            full((DS, DS // 4)), full((1, DS // 4)),
            full((HL, HG)), full((DS // 4, HG)),
        ],
        out_specs=[
            pl.BlockSpec((B, R2, HG), lambda i: (_I0, i, _I0)),
            pl.BlockSpec((R2, 1), lambda i: (i, _I0)),
        ],
        out_shape=[
            jax.ShapeDtypeStruct((B, N, HG), F32),
            jax.ShapeDtypeStruct((N, 1), F32),
        ],
    )(deg2, comb3, sg, wspT, bsp2, w1hT, w1pT)


def _mid_body(agg_r, h_r, dinv_r, b_r, w2_r, out_ref):
    dinv = dinv_r[...]
    for b in range(B):
        x = jnp.maximum(dinv * (agg_r[b] + h_r[b]) + b_r[...], 0.0)
        out_ref[b] = dinv * jnp.dot(x, w2_r[...], preferred_element_type=F32)


def _gcn_mid(agg1, h1s, dinv, b1v, w2T):
    full = lambda shp: pl.BlockSpec(shp, lambda i: tuple(_I0 for _ in shp))
    return pl.pallas_call(
        _mid_body,
        grid=(N // R2,),
        in_specs=[
            pl.BlockSpec((B, R2, HG), lambda i: (_I0, i, _I0)),
            pl.BlockSpec((B, R2, HG), lambda i: (_I0, i, _I0)),
            pl.BlockSpec((R2, 1), lambda i: (i, _I0)),
            full((1, HG)), full((HG, HG)),
        ],
        out_specs=pl.BlockSpec((B, R2, HG), lambda i: (_I0, i, _I0)),
        out_shape=jax.ShapeDtypeStruct((B, N, HG), F32),
    )(agg1, h1s, dinv, b1v, w2T)


def _fin_body(agg_r, h_r, dinv_r, b_r, wl_r, bl_r, out_ref):
    dinv = dinv_r[...]
    for b in range(B):
        x = jnp.maximum(dinv * (agg_r[b] + h_r[b]) + b_r[...], 0.0)
        out_ref[b] = jnp.dot(x, wl_r[...], preferred_element_type=F32) + bl_r[...]


def _gcn_fin(agg2, h2s, dinv, b2v, wlT, blv):
    full = lambda shp: pl.BlockSpec(shp, lambda i: tuple(_I0 for _ in shp))
    return pl.pallas_call(
        _fin_body,
        grid=(N // R2,),
        in_specs=[
            pl.BlockSpec((B, R2, HG), lambda i: (_I0, i, _I0)),
            pl.BlockSpec((B, R2, HG), lambda i: (_I0, i, _I0)),
            pl.BlockSpec((R2, 1), lambda i: (i, _I0)),
            full((1, HG)), full((HG, 1)), full((1, 1)),
        ],
        out_specs=pl.BlockSpec((B, R2, 1), lambda i: (_I0, i, _I0)),
        out_shape=jax.ShapeDtypeStruct((B, N, 1), F32),
    )(agg2, h2s, dinv, b2v, wlT, blv)


_EPS = EPAD // 16
_EPW = EPAD // 32
_ZR = NDUM // 16


_NCH = _EPS // CHUNK
_GK = 2


def _make_agg_body(width):
    def _agg_body(h_hbm, srcs_hbm, dst_hbm, zeros_hbm, out_hbm,
                  src_v, dstb, bufs, acc_s, gsem, ssem):
        c = lax.axis_index("c")
        s = lax.axis_index("s")
        pltpu.sync_copy(zeros_hbm.at[pl.ds(s * _ZR, _ZR)],
                        acc_s.at[pl.ds(s * _ZR, _ZR)])
        pltpu.sync_copy(srcs_hbm.at[c, pl.ds(s * _NCH, _NCH)], src_v)
        dbase = s * _EPS
        plsc.subcore_barrier()

        @pl.loop(jnp.int32(0), jnp.int32(_NCH), step=jnp.int32(_GK))
        def _group(i0):
            for j in range(_GK):
                pltpu.make_async_copy(
                    dst_hbm.at[pl.ds(dbase + (i0 + np.int32(j)) * CHUNK,
                                     CHUNK)],
                    dstb.at[np.int32(j)], gsem).start()
                pltpu.make_async_copy(h_hbm.at[src_v.at[i0 + np.int32(j)]],
                                      bufs.at[np.int32(j)], gsem).start()
            for j in range(_GK):
                pltpu.make_async_copy(
                    dst_hbm.at[pl.ds(dbase + (i0 + np.int32(j)) * CHUNK,
                                     CHUNK)],
                    dstb.at[np.int32(j)], gsem).wait()
                pltpu.make_async_copy(h_hbm.at[src_v.at[i0 + np.int32(j)]],
                                      bufs.at[np.int32(j)], gsem).wait()
            for j in range(_GK):
                pltpu.make_async_copy(bufs.at[np.int32(j)],
                                      acc_s.at[dstb.at[np.int32(j)]],
                                      ssem).start(add=True)
            for j in range(_GK):
                pltpu.make_async_copy(bufs.at[np.int32(j)],
                                      acc_s.at[dstb.at[np.int32(j)]],
                                      ssem).wait()

        plsc.subcore_barrier()
        pltpu.sync_copy(acc_s.at[pl.ds(s * _ZR, _ZR)],
                        out_hbm.at[c, pl.ds(s * _ZR, _ZR)])
    return _agg_body


@functools.lru_cache(maxsize=None)
def _build_agg_kernel(width):
    mesh = plsc.VectorSubcoreMesh(core_axis_name="c", subcore_axis_name="s")
    return functools.partial(
        pl.kernel,
        mesh=mesh,
        out_type=jax.ShapeDtypeStruct((B, NDUM, width), F32),
        scratch_types=[
            pltpu.VMEM((_NCH, CHUNK), jnp.int32),
            pltpu.VMEM((_GK, CHUNK), jnp.int32),
            pltpu.VMEM((_GK, CHUNK, width), F32),
            pltpu.VMEM_SHARED((NDUM, width), F32),
            pltpu.SemaphoreType.DMA,
            pltpu.SemaphoreType.DMA,
        ],
    )(_make_agg_body(width))


_DGK = 8


def _deg_body(ones_hbm, dst_hbm, zeros_hbm, out_hbm, dstb, ones_v, acc_s,
              gsem, ssem):
    c = lax.axis_index("c")
    s = lax.axis_index("s")
    pltpu.sync_copy(zeros_hbm.at[pl.ds(s * _ZR, _ZR)],
                    acc_s.at[pl.ds(s * _ZR, _ZR)])
    pltpu.sync_copy(ones_hbm, ones_v)
    dbase = s * _EPS
    plsc.subcore_barrier()

    @pl.loop(jnp.int32(0), jnp.int32(_NCH), step=jnp.int32(_DGK))
    def _group(i0):
        for j in range(_DGK):
            pltpu.make_async_copy(
                dst_hbm.at[pl.ds(dbase + (i0 + np.int32(j)) * CHUNK, CHUNK)],
                dstb.at[np.int32(j)], gsem).start()
        for j in range(_DGK):
            pltpu.make_async_copy(
                dst_hbm.at[pl.ds(dbase + (i0 + np.int32(j)) * CHUNK, CHUNK)],
                dstb.at[np.int32(j)], gsem).wait()
        for j in range(_DGK):
            pltpu.make_async_copy(ones_v, acc_s.at[dstb.at[np.int32(j)]],
                                  ssem).start(add=True)
        for j in range(_DGK):
            pltpu.make_async_copy(ones_v, acc_s.at[dstb.at[np.int32(j)]],
                                  ssem).wait()

    plsc.subcore_barrier()
    pltpu.sync_copy(acc_s.at[pl.ds(s * _ZR, _ZR)],
                    out_hbm.at[c, pl.ds(s * _ZR, _ZR)])


@functools.lru_cache(maxsize=None)
def _build_deg_kernel():
    mesh = plsc.VectorSubcoreMesh(core_axis_name="c", subcore_axis_name="s")
    return functools.partial(
        pl.kernel,
        mesh=mesh,
        out_type=jax.ShapeDtypeStruct((B, NDUM, HG), F32),
        scratch_types=[
            pltpu.VMEM((_DGK, CHUNK), jnp.int32),
            pltpu.VMEM((CHUNK, HG), F32),
            pltpu.VMEM_SHARED((NDUM, HG), F32),
            pltpu.SemaphoreType.DMA,
            pltpu.SemaphoreType.DMA,
        ],
    )(_deg_body)


def _deg_call(ones_chunk, dstp, zeros128):
    return _build_deg_kernel()(ones_chunk, dstp, zeros128)[:, :N]


def _agg_call(hflat, srcs2, dstp, zeros128):
    return _build_agg_kernel(HG)(hflat, srcs2, dstp, zeros128)[:, :N]


def kernel(dynamic_features, static_features, edge_index, W_ih0, W_hh0, b_ih0,
           b_hh0, W_ih1, W_hh1, b_ih1, b_hh1, Ws, bs, Wc, bc, Wsp, bsp,
           W1, b1, W2, b2, Wl, bl):
    x2d = dynamic_features.reshape(B * N, T * DD)
    xs = static_features.reshape(B * N, DS)
    sg = static_features[0]

    src = edge_index[0].astype(jnp.int32)
    dst = edge_index[1].astype(jnp.int32)
    pad = EPAD - E
    srcp = jnp.concatenate([src, jnp.zeros((pad,), jnp.int32)])
    dstp = jnp.concatenate([dst, jnp.full((pad,), N, jnp.int32)])
    srcs2 = jnp.stack([srcp, srcp + N]).reshape(B, EPAD // CHUNK, CHUNK)
    zeros128 = jnp.zeros((NDUM, HG), F32)
    ones_chunk = jnp.ones((CHUNK, HG), F32)

    b0 = (b_ih0 + b_hh0).reshape(1, 4 * HL)
    b1s = (b_ih1 + b_hh1).reshape(1, 4 * HL)
    comb = _encode(x2d, xs, W_ih0.T, W_hh0.T, b0, W_ih1.T, W_hh1.T, b1s,
                   Ws.T, bs.reshape(1, -1), Wc[:, :HL].T, Wc[:, HL:].T,
                   bc.reshape(1, -1))

    deg2 = _deg_call(ones_chunk, dstp, zeros128)

    h1s, dinv = _gcn_pre(deg2, comb.reshape(B, N, HL), sg, Wsp.T,
                         bsp.reshape(1, -1), W1[:, :HL].T, W1[:, HL:].T)

    agg1 = _agg_call(h1s.reshape(B * N, HG), srcs2, dstp, zeros128)

    h2s = _gcn_mid(agg1, h1s, dinv, b1.reshape(1, -1), W2.T)

    agg2 = _agg_call(h2s.reshape(B * N, HG), srcs2, dstp, zeros128)

    res = _gcn_fin(agg2, h2s, dinv, b2.reshape(1, -1), Wl.T, bl.reshape(1, 1))
    return res[:, :, 0]


# speedup vs baseline: 6.7990x; 1.0612x over previous
import functools

import numpy as np

import jax
import jax.numpy as jnp
from jax import lax
from jax.experimental import pallas as pl
from jax.experimental.pallas import tpu as pltpu
from jax.experimental.pallas import tpu_sc as plsc

F32 = jnp.float32
_I0 = np.int32(0)
B, N, T, DD, DS, HL, HG, E = 2, 10000, 24, 16, 32, 128, 128, 160000
EPAD = 163840
NDUM = 10240
R1 = 400
R2 = 1000
CHUNK = 128


def _enc_body(x_ref, xs_ref, wih0_r, whh0_r, b0_r, wih1_r, whh1_r, b1_r,
              ws_r, bs_r, wch_r, wcs_r, bc_r, out_ref):
    wih0 = wih0_r[...]
    whh0 = whh0_r[...]
    wih1 = wih1_r[...]
    whh1 = whh1_r[...]
    b0 = b0_r[...]
    b1 = b1_r[...]
    h0 = jnp.zeros((R1, HL), F32)
    c0 = jnp.zeros((R1, HL), F32)
    h1 = jnp.zeros((R1, HL), F32)
    c1 = jnp.zeros((R1, HL), F32)
    for t in range(T):
        xt = x_ref[:, t * DD:(t + 1) * DD]
        g = (jnp.dot(xt, wih0, preferred_element_type=F32)
             + jnp.dot(h0, whh0, preferred_element_type=F32) + b0)
        ig = jax.nn.sigmoid(g[:, 0:HL])
        fg = jax.nn.sigmoid(g[:, HL:2 * HL])
        gg = jnp.tanh(g[:, 2 * HL:3 * HL])
        og = jax.nn.sigmoid(g[:, 3 * HL:4 * HL])
        c0 = fg * c0 + ig * gg
        h0 = og * jnp.tanh(c0)
        g = (jnp.dot(h0, wih1, preferred_element_type=F32)
             + jnp.dot(h1, whh1, preferred_element_type=F32) + b1)
        ig = jax.nn.sigmoid(g[:, 0:HL])
        fg = jax.nn.sigmoid(g[:, HL:2 * HL])
        gg = jnp.tanh(g[:, 2 * HL:3 * HL])
        og = jax.nn.sigmoid(g[:, 3 * HL:4 * HL])
        c1 = fg * c1 + ig * gg
        h1 = og * jnp.tanh(c1)
    s = jnp.maximum(jnp.dot(xs_ref[...], ws_r[...],
                            preferred_element_type=F32) + bs_r[...], 0.0)
    comb = jnp.maximum(jnp.dot(h1, wch_r[...], preferred_element_type=F32)
                       + jnp.dot(s, wcs_r[...], preferred_element_type=F32)
                       + bc_r[...], 0.0)
    out_ref[...] = comb


def _encode(x2d, xs, wih0T, whh0T, b0, wih1T, whh1T, b1s, wsT, bs2,
            wchT, wcsT, bc2):
    full = lambda shp: pl.BlockSpec(shp, lambda i: tuple(_I0 for _ in shp))
    return pl.pallas_call(
        _enc_body,
        grid=(B * N // R1,),
        in_specs=[
            pl.BlockSpec((R1, T * DD), lambda i: (i, _I0)),
            pl.BlockSpec((R1, DS), lambda i: (i, _I0)),
            full((DD, 4 * HL)), full((HL, 4 * HL)), full((1, 4 * HL)),
            full((HL, 4 * HL)), full((HL, 4 * HL)), full((1, 4 * HL)),
            full((DS, HL // 2)), full((1, HL // 2)),
            full((HL, HL)), full((HL // 2, HL)), full((1, HL)),
        ],
        out_specs=pl.BlockSpec((R1, HL), lambda i: (i, _I0)),
        out_shape=jax.ShapeDtypeStruct((B * N, HL), F32),
    )(x2d, xs, wih0T, whh0T, b0, wih1T, whh1T, b1s, wsT, bs2, wchT, wcsT, bc2)


def _pre_body(deg_r, comb_r, sg_r, wsp_r, bsp_r, w1h_r, w1p_r,
              h1_ref, dinv_ref):
    d = deg_r[0][:, 0:1] + 1.0
    dinv = lax.rsqrt(d)
    dinv_ref[...] = dinv
    ps = jnp.maximum(jnp.dot(sg_r[...], wsp_r[...],
                             preferred_element_type=F32) + bsp_r[...], 0.0)
    pw = jnp.dot(ps, w1p_r[...], preferred_element_type=F32)
    for b in range(B):
        h = jnp.dot(comb_r[b], w1h_r[...], preferred_element_type=F32) + pw
        h1_ref[b] = dinv * h


def _gcn_pre(deg2, comb3, sg, wspT, bsp2, w1hT, w1pT):
    full = lambda shp: pl.BlockSpec(shp, lambda i: tuple(_I0 for _ in shp))
    return pl.pallas_call(
        _pre_body,
        grid=(N // R2,),
        in_specs=[
            pl.BlockSpec((B, R2, HG), lambda i: (_I0, i, _I0)),
            pl.BlockSpec((B, R2, HL), lambda i: (_I0, i, _I0)),
            pl.BlockSpec((R2, DS), lambda i: (i, _I0)),
            full((DS, DS // 4)), full((1, DS // 4)),
            full((HL, HG)), full((DS // 4, HG)),
        ],
        out_specs=[
            pl.BlockSpec((B, R2, HG), lambda i: (_I0, i, _I0)),
            pl.BlockSpec((R2, 1), lambda i: (i, _I0)),
        ],
        out_shape=[
            jax.ShapeDtypeStruct((B, N, HG), F32),
            jax.ShapeDtypeStruct((N, 1), F32),
        ],
    )(deg2, comb3, sg, wspT, bsp2, w1hT, w1pT)


def _mid_body(agg_r, h_r, dinv_r, b_r, w2_r, out_ref):
    dinv = dinv_r[...]
    for b in range(B):
        x = jnp.maximum(dinv * (agg_r[b] + h_r[b]) + b_r[...], 0.0)
        out_ref[b] = dinv * jnp.dot(x, w2_r[...], preferred_element_type=F32)


def _gcn_mid(agg1, h1s, dinv, b1v, w2T):
    full = lambda shp: pl.BlockSpec(shp, lambda i: tuple(_I0 for _ in shp))
    return pl.pallas_call(
        _mid_body,
        grid=(N // R2,),
        in_specs=[
            pl.BlockSpec((B, R2, HG), lambda i: (_I0, i, _I0)),
            pl.BlockSpec((B, R2, HG), lambda i: (_I0, i, _I0)),
            pl.BlockSpec((R2, 1), lambda i: (i, _I0)),
            full((1, HG)), full((HG, HG)),
        ],
        out_specs=pl.BlockSpec((B, R2, HG), lambda i: (_I0, i, _I0)),
        out_shape=jax.ShapeDtypeStruct((B, N, HG), F32),
    )(agg1, h1s, dinv, b1v, w2T)


def _fin_body(agg_r, h_r, dinv_r, b_r, wl_r, bl_r, out_ref):
    dinv = dinv_r[...]
    for b in range(B):
        x = jnp.maximum(dinv * (agg_r[b] + h_r[b]) + b_r[...], 0.0)
        out_ref[b] = jnp.dot(x, wl_r[...], preferred_element_type=F32) + bl_r[...]


def _gcn_fin(agg2, h2s, dinv, b2v, wlT, blv):
    full = lambda shp: pl.BlockSpec(shp, lambda i: tuple(_I0 for _ in shp))
    return pl.pallas_call(
        _fin_body,
        grid=(N // R2,),
        in_specs=[
            pl.BlockSpec((B, R2, HG), lambda i: (_I0, i, _I0)),
            pl.BlockSpec((B, R2, HG), lambda i: (_I0, i, _I0)),
            pl.BlockSpec((R2, 1), lambda i: (i, _I0)),
            full((1, HG)), full((HG, 1)), full((1, 1)),
        ],
        out_specs=pl.BlockSpec((B, R2, 1), lambda i: (_I0, i, _I0)),
        out_shape=jax.ShapeDtypeStruct((B, N, 1), F32),
    )(agg2, h2s, dinv, b2v, wlT, blv)


_EPS = EPAD // 16
_EPW = EPAD // 32
_ZR = NDUM // 16


_NCH = _EPS // CHUNK
_GK = 2


def _make_agg_body(width):
    def _agg_body(h_hbm, srcs_hbm, dst_hbm, zeros_hbm, out_hbm,
                  src_v, dstb, bufs, acc_s, gsem0, gsem1, ssem0, ssem1):
        c = lax.axis_index("c")
        s = lax.axis_index("s")
        gsems = [gsem0, gsem1]
        ssems = [ssem0, ssem1]
        pltpu.sync_copy(zeros_hbm.at[pl.ds(s * _ZR, _ZR)],
                        acc_s.at[pl.ds(s * _ZR, _ZR)])
        pltpu.sync_copy(srcs_hbm.at[c, pl.ds(s * _NCH, _NCH)], src_v)
        dbase = s * _EPS
        plsc.subcore_barrier()

        def idx_cp(i, j):
            return pltpu.make_async_copy(
                dst_hbm.at[pl.ds(dbase + i * CHUNK, CHUNK)],
                dstb.at[np.int32(j)], gsems[j])

        def gath(i, j):
            return pltpu.make_async_copy(
                h_hbm.at[src_v.at[i]], bufs.at[np.int32(j)], gsems[j])

        def scat(j):
            return pltpu.make_async_copy(
                bufs.at[np.int32(j)], acc_s.at[dstb.at[np.int32(j)]],
                ssems[j])

        for j in range(2):
            idx_cp(jnp.int32(j), j).start()
            gath(jnp.int32(j), j).start()

        @pl.loop(jnp.int32(0), jnp.int32(_NCH - 2), step=jnp.int32(2))
        def _pipe(i0):
            for j in range(2):
                i = i0 + np.int32(j)
                idx_cp(i, j).wait()
                gath(i, j).wait()
                scat(j).start(add=True)
                scat(j).wait()
                idx_cp(i + np.int32(2), j).start()
                gath(i + np.int32(2), j).start()

        for j in range(2):
            i = jnp.int32(_NCH - 2 + j)
            idx_cp(i, j).wait()
            gath(i, j).wait()
            scat(j).start(add=True)
            scat(j).wait()

        plsc.subcore_barrier()
        pltpu.sync_copy(acc_s.at[pl.ds(s * _ZR, _ZR)],
                        out_hbm.at[c, pl.ds(s * _ZR, _ZR)])
    return _agg_body


@functools.lru_cache(maxsize=None)
def _build_agg_kernel(width):
    mesh = plsc.VectorSubcoreMesh(core_axis_name="c", subcore_axis_name="s")
    return functools.partial(
        pl.kernel,
        mesh=mesh,
        out_type=jax.ShapeDtypeStruct((B, NDUM, width), F32),
        scratch_types=[
            pltpu.VMEM((_NCH, CHUNK), jnp.int32),
            pltpu.VMEM((_GK, CHUNK), jnp.int32),
            pltpu.VMEM((_GK, CHUNK, width), F32),
            pltpu.VMEM_SHARED((NDUM, width), F32),
            pltpu.SemaphoreType.DMA,
            pltpu.SemaphoreType.DMA,
            pltpu.SemaphoreType.DMA,
            pltpu.SemaphoreType.DMA,
        ],
    )(_make_agg_body(width))


_DGK = 8


def _deg_body(ones_hbm, dst_hbm, zeros_hbm, out_hbm, dstb, ones_v, acc_s,
              gsem, ssem):
    c = lax.axis_index("c")
    s = lax.axis_index("s")
    pltpu.sync_copy(zeros_hbm.at[pl.ds(s * _ZR, _ZR)],
                    acc_s.at[pl.ds(s * _ZR, _ZR)])
    pltpu.sync_copy(ones_hbm, ones_v)
    dbase = s * _EPS
    plsc.subcore_barrier()

    @pl.loop(jnp.int32(0), jnp.int32(_NCH), step=jnp.int32(_DGK))
    def _group(i0):
        for j in range(_DGK):
            pltpu.make_async_copy(
                dst_hbm.at[pl.ds(dbase + (i0 + np.int32(j)) * CHUNK, CHUNK)],
                dstb.at[np.int32(j)], gsem).start()
        for j in range(_DGK):
            pltpu.make_async_copy(
                dst_hbm.at[pl.ds(dbase + (i0 + np.int32(j)) * CHUNK, CHUNK)],
                dstb.at[np.int32(j)], gsem).wait()
        for j in range(_DGK):
            pltpu.make_async_copy(ones_v, acc_s.at[dstb.at[np.int32(j)]],
                                  ssem).start(add=True)
        for j in range(_DGK):
            pltpu.make_async_copy(ones_v, acc_s.at[dstb.at[np.int32(j)]],
                                  ssem).wait()

    plsc.subcore_barrier()
    pltpu.sync_copy(acc_s.at[pl.ds(s * _ZR, _ZR)],
                    out_hbm.at[c, pl.ds(s * _ZR, _ZR)])


@functools.lru_cache(maxsize=None)
def _build_deg_kernel():
    mesh = plsc.VectorSubcoreMesh(core_axis_name="c", subcore_axis_name="s")
    return functools.partial(
        pl.kernel,
        mesh=mesh,
        out_type=jax.ShapeDtypeStruct((B, NDUM, HG), F32),
        scratch_types=[
            pltpu.VMEM((_DGK, CHUNK), jnp.int32),
            pltpu.VMEM((CHUNK, HG), F32),
            pltpu.VMEM_SHARED((NDUM, HG), F32),
            pltpu.SemaphoreType.DMA,
            pltpu.SemaphoreType.DMA,
        ],
    )(_deg_body)


def _deg_call(ones_chunk, dstp, zeros128):
    return _build_deg_kernel()(ones_chunk, dstp, zeros128)[:, :N]


def _agg_call(hflat, srcs2, dstp, zeros128):
    return _build_agg_kernel(HG)(hflat, srcs2, dstp, zeros128)[:, :N]


def kernel(dynamic_features, static_features, edge_index, W_ih0, W_hh0, b_ih0,
           b_hh0, W_ih1, W_hh1, b_ih1, b_hh1, Ws, bs, Wc, bc, Wsp, bsp,
           W1, b1, W2, b2, Wl, bl):
    x2d = dynamic_features.reshape(B * N, T * DD)
    xs = static_features.reshape(B * N, DS)
    sg = static_features[0]

    src = edge_index[0].astype(jnp.int32)
    dst = edge_index[1].astype(jnp.int32)
    pad = EPAD - E
    srcp = jnp.concatenate([src, jnp.zeros((pad,), jnp.int32)])
    dstp = jnp.concatenate([dst, jnp.full((pad,), N, jnp.int32)])
    srcs2 = jnp.stack([srcp, srcp + N]).reshape(B, EPAD // CHUNK, CHUNK)
    zeros128 = jnp.zeros((NDUM, HG), F32)
    ones_chunk = jnp.ones((CHUNK, HG), F32)

    b0 = (b_ih0 + b_hh0).reshape(1, 4 * HL)
    b1s = (b_ih1 + b_hh1).reshape(1, 4 * HL)
    comb = _encode(x2d, xs, W_ih0.T, W_hh0.T, b0, W_ih1.T, W_hh1.T, b1s,
                   Ws.T, bs.reshape(1, -1), Wc[:, :HL].T, Wc[:, HL:].T,
                   bc.reshape(1, -1))

    deg2 = _deg_call(ones_chunk, dstp, zeros128)

    h1s, dinv = _gcn_pre(deg2, comb.reshape(B, N, HL), sg, Wsp.T,
                         bsp.reshape(1, -1), W1[:, :HL].T, W1[:, HL:].T)

    agg1 = _agg_call(h1s.reshape(B * N, HG), srcs2, dstp, zeros128)

    h2s = _gcn_mid(agg1, h1s, dinv, b1.reshape(1, -1), W2.T)

    agg2 = _agg_call(h2s.reshape(B * N, HG), srcs2, dstp, zeros128)

    res = _gcn_fin(agg2, h2s, dinv, b2.reshape(1, -1), Wl.T, bl.reshape(1, 1))
    return res[:, :, 0]


# speedup vs baseline: 7.2434x; 1.0654x over previous
import functools

import numpy as np

import jax
import jax.numpy as jnp
from jax import lax
from jax.experimental import pallas as pl
from jax.experimental.pallas import tpu as pltpu
from jax.experimental.pallas import tpu_sc as plsc

F32 = jnp.float32
_I0 = np.int32(0)
B, N, T, DD, DS, HL, HG, E = 2, 10000, 24, 16, 32, 128, 128, 160000
EPAD = 163840
NDUM = 10240
R1 = 400
R2 = 1000
CHUNK = 128


def _sigmoid(x):
    return 0.5 * jnp.tanh(0.5 * x) + 0.5


def _enc_body(x_ref, xs_ref, wih0_r, whh0_r, b0_r, wih1_r, whh1_r, b1_r,
              ws_r, bs_r, wch_r, wcs_r, bc_r, out_ref):
    wih0 = wih0_r[...]
    whh0 = whh0_r[...]
    wih1 = wih1_r[...]
    whh1 = whh1_r[...]
    b0 = b0_r[...]
    b1 = b1_r[...]
    h0 = jnp.zeros((R1, HL), F32)
    c0 = jnp.zeros((R1, HL), F32)
    h1 = jnp.zeros((R1, HL), F32)
    c1 = jnp.zeros((R1, HL), F32)
    for t in range(T):
        xt = x_ref[:, t * DD:(t + 1) * DD]
        g = (jnp.dot(xt, wih0, preferred_element_type=F32)
             + jnp.dot(h0, whh0, preferred_element_type=F32) + b0)
        ig = _sigmoid(g[:, 0:HL])
        fg = _sigmoid(g[:, HL:2 * HL])
        gg = jnp.tanh(g[:, 2 * HL:3 * HL])
        og = _sigmoid(g[:, 3 * HL:4 * HL])
        c0 = fg * c0 + ig * gg
        h0 = og * jnp.tanh(c0)
        g = (jnp.dot(h0, wih1, preferred_element_type=F32)
             + jnp.dot(h1, whh1, preferred_element_type=F32) + b1)
        ig = _sigmoid(g[:, 0:HL])
        fg = _sigmoid(g[:, HL:2 * HL])
        gg = jnp.tanh(g[:, 2 * HL:3 * HL])
        og = _sigmoid(g[:, 3 * HL:4 * HL])
        c1 = fg * c1 + ig * gg
        h1 = og * jnp.tanh(c1)
    s = jnp.maximum(jnp.dot(xs_ref[...], ws_r[...],
                            preferred_element_type=F32) + bs_r[...], 0.0)
    comb = jnp.maximum(jnp.dot(h1, wch_r[...], preferred_element_type=F32)
                       + jnp.dot(s, wcs_r[...], preferred_element_type=F32)
                       + bc_r[...], 0.0)
    out_ref[...] = comb


def _encode(x2d, xs, wih0T, whh0T, b0, wih1T, whh1T, b1s, wsT, bs2,
            wchT, wcsT, bc2):
    full = lambda shp: pl.BlockSpec(shp, lambda i: tuple(_I0 for _ in shp))
    return pl.pallas_call(
        _enc_body,
        grid=(B * N // R1,),
        in_specs=[
            pl.BlockSpec((R1, T * DD), lambda i: (i, _I0)),
            pl.BlockSpec((R1, DS), lambda i: (i, _I0)),
            full((DD, 4 * HL)), full((HL, 4 * HL)), full((1, 4 * HL)),
            full((HL, 4 * HL)), full((HL, 4 * HL)), full((1, 4 * HL)),
            full((DS, HL // 2)), full((1, HL // 2)),
            full((HL, HL)), full((HL // 2, HL)), full((1, HL)),
        ],
        out_specs=pl.BlockSpec((R1, HL), lambda i: (i, _I0)),
        out_shape=jax.ShapeDtypeStruct((B * N, HL), F32),
    )(x2d, xs, wih0T, whh0T, b0, wih1T, whh1T, b1s, wsT, bs2, wchT, wcsT, bc2)


def _pre_body(deg_r, comb_r, sg_r, wsp_r, bsp_r, w1h_r, w1p_r,
              h1_ref, dinv_ref):
    d = deg_r[0][:, 0:1] + 1.0
    dinv = lax.rsqrt(d)
    dinv_ref[...] = dinv
    ps = jnp.maximum(jnp.dot(sg_r[...], wsp_r[...],
                             preferred_element_type=F32) + bsp_r[...], 0.0)
    pw = jnp.dot(ps, w1p_r[...], preferred_element_type=F32)
    for b in range(B):
        h = jnp.dot(comb_r[b], w1h_r[...], preferred_element_type=F32) + pw
        h1_ref[b] = dinv * h


def _gcn_pre(deg2, comb3, sg, wspT, bsp2, w1hT, w1pT):
    full = lambda shp: pl.BlockSpec(shp, lambda i: tuple(_I0 for _ in shp))
    return pl.pallas_call(
        _pre_body,
        grid=(N // R2,),
        in_specs=[
            pl.BlockSpec((B, R2, HG), lambda i: (_I0, i, _I0)),
            pl.BlockSpec((B, R2, HL), lambda i: (_I0, i, _I0)),
            pl.BlockSpec((R2, DS), lambda i: (i, _I0)),
            full((DS, DS // 4)), full((1, DS // 4)),
            full((HL, HG)), full((DS // 4, HG)),
        ],
        out_specs=[
            pl.BlockSpec((B, R2, HG), lambda i: (_I0, i, _I0)),
            pl.BlockSpec((R2, 1), lambda i: (i, _I0)),
        ],
        out_shape=[
            jax.ShapeDtypeStruct((B, N, HG), F32),
            jax.ShapeDtypeStruct((N, 1), F32),
        ],
    )(deg2, comb3, sg, wspT, bsp2, w1hT, w1pT)


def _mid_body(agg_r, h_r, dinv_r, b_r, w2_r, out_ref):
    dinv = dinv_r[...]
    for b in range(B):
        x = jnp.maximum(dinv * (agg_r[b] + h_r[b]) + b_r[...], 0.0)
        out_ref[b] = dinv * jnp.dot(x, w2_r[...], preferred_element_type=F32)


def _gcn_mid(agg1, h1s, dinv, b1v, w2T):
    full = lambda shp: pl.BlockSpec(shp, lambda i: tuple(_I0 for _ in shp))
    return pl.pallas_call(
        _mid_body,
        grid=(N // R2,),
        in_specs=[
            pl.BlockSpec((B, R2, HG), lambda i: (_I0, i, _I0)),
            pl.BlockSpec((B, R2, HG), lambda i: (_I0, i, _I0)),
            pl.BlockSpec((R2, 1), lambda i: (i, _I0)),
            full((1, HG)), full((HG, HG)),
        ],
        out_specs=pl.BlockSpec((B, R2, HG), lambda i: (_I0, i, _I0)),
        out_shape=jax.ShapeDtypeStruct((B, N, HG), F32),
    )(agg1, h1s, dinv, b1v, w2T)


def _fin_body(agg_r, h_r, dinv_r, b_r, wl_r, bl_r, out_ref):
    dinv = dinv_r[...]
    for b in range(B):
        x = jnp.maximum(dinv * (agg_r[b] + h_r[b]) + b_r[...], 0.0)
        out_ref[b] = jnp.dot(x, wl_r[...], preferred_element_type=F32) + bl_r[...]


def _gcn_fin(agg2, h2s, dinv, b2v, wlT, blv):
    full = lambda shp: pl.BlockSpec(shp, lambda i: tuple(_I0 for _ in shp))
    return pl.pallas_call(
        _fin_body,
        grid=(N // R2,),
        in_specs=[
            pl.BlockSpec((B, R2, HG), lambda i: (_I0, i, _I0)),
            pl.BlockSpec((B, R2, HG), lambda i: (_I0, i, _I0)),
            pl.BlockSpec((R2, 1), lambda i: (i, _I0)),
            full((1, HG)), full((HG, 1)), full((1, 1)),
        ],
        out_specs=pl.BlockSpec((B, R2, 1), lambda i: (_I0, i, _I0)),
        out_shape=jax.ShapeDtypeStruct((B, N, 1), F32),
    )(agg2, h2s, dinv, b2v, wlT, blv)


_EPS = EPAD // 16
_EPW = EPAD // 32
_ZR = NDUM // 16


_NCH = _EPS // CHUNK
_GK = 2


def _make_agg_body(width):
    def _agg_body(h_hbm, srcs_hbm, dst_hbm, zeros_hbm, out_hbm,
                  src_v, dstb, bufs, acc_s, gsem0, gsem1, ssem0, ssem1):
        c = lax.axis_index("c")
        s = lax.axis_index("s")
        gsems = [gsem0, gsem1]
        ssems = [ssem0, ssem1]
        pltpu.sync_copy(zeros_hbm.at[pl.ds(s * _ZR, _ZR)],
                        acc_s.at[pl.ds(s * _ZR, _ZR)])
        pltpu.sync_copy(srcs_hbm.at[c, pl.ds(s * _NCH, _NCH)], src_v)
        dbase = s * _EPS
        plsc.subcore_barrier()

        def idx_cp(i, j):
            return pltpu.make_async_copy(
                dst_hbm.at[pl.ds(dbase + i * CHUNK, CHUNK)],
                dstb.at[np.int32(j)], gsems[j])

        def gath(i, j):
            return pltpu.make_async_copy(
                h_hbm.at[src_v.at[i]], bufs.at[np.int32(j)], gsems[j])

        def scat(j):
            return pltpu.make_async_copy(
                bufs.at[np.int32(j)], acc_s.at[dstb.at[np.int32(j)]],
                ssems[j])

        for j in range(2):
            idx_cp(jnp.int32(j), j).start()
            gath(jnp.int32(j), j).start()

        @pl.loop(jnp.int32(0), jnp.int32(_NCH - 2), step=jnp.int32(2))
        def _pipe(i0):
            for j in range(2):
                i = i0 + np.int32(j)
                idx_cp(i, j).wait()
                gath(i, j).wait()
                scat(j).start(add=True)
                scat(j).wait()
                idx_cp(i + np.int32(2), j).start()
                gath(i + np.int32(2), j).start()

        for j in range(2):
            i = jnp.int32(_NCH - 2 + j)
            idx_cp(i, j).wait()
            gath(i, j).wait()
            scat(j).start(add=True)
            scat(j).wait()

        plsc.subcore_barrier()
        pltpu.sync_copy(acc_s.at[pl.ds(s * _ZR, _ZR)],
                        out_hbm.at[c, pl.ds(s * _ZR, _ZR)])
    return _agg_body


@functools.lru_cache(maxsize=None)
def _build_agg_kernel(width):
    mesh = plsc.VectorSubcoreMesh(core_axis_name="c", subcore_axis_name="s")
    return functools.partial(
        pl.kernel,
        mesh=mesh,
        out_type=jax.ShapeDtypeStruct((B, NDUM, width), F32),
        scratch_types=[
            pltpu.VMEM((_NCH, CHUNK), jnp.int32),
            pltpu.VMEM((_GK, CHUNK), jnp.int32),
            pltpu.VMEM((_GK, CHUNK, width), F32),
            pltpu.VMEM_SHARED((NDUM, width), F32),
            pltpu.SemaphoreType.DMA,
            pltpu.SemaphoreType.DMA,
            pltpu.SemaphoreType.DMA,
            pltpu.SemaphoreType.DMA,
        ],
    )(_make_agg_body(width))


_DGK = 8


def _deg_body(ones_hbm, dst_hbm, zeros_hbm, out_hbm, dstb, ones_v, acc_s,
              gsem, ssem):
    c = lax.axis_index("c")
    s = lax.axis_index("s")
    pltpu.sync_copy(zeros_hbm.at[pl.ds(s * _ZR, _ZR)],
                    acc_s.at[pl.ds(s * _ZR, _ZR)])
    pltpu.sync_copy(ones_hbm, ones_v)
    dbase = s * _EPS
    plsc.subcore_barrier()

    @pl.loop(jnp.int32(0), jnp.int32(_NCH), step=jnp.int32(_DGK))
    def _group(i0):
        for j in range(_DGK):
            pltpu.make_async_copy(
                dst_hbm.at[pl.ds(dbase + (i0 + np.int32(j)) * CHUNK, CHUNK)],
                dstb.at[np.int32(j)], gsem).start()
        for j in range(_DGK):
            pltpu.make_async_copy(
                dst_hbm.at[pl.ds(dbase + (i0 + np.int32(j)) * CHUNK, CHUNK)],
                dstb.at[np.int32(j)], gsem).wait()
        for j in range(_DGK):
            pltpu.make_async_copy(ones_v, acc_s.at[dstb.at[np.int32(j)]],
                                  ssem).start(add=True)
        for j in range(_DGK):
            pltpu.make_async_copy(ones_v, acc_s.at[dstb.at[np.int32(j)]],
                                  ssem).wait()

    plsc.subcore_barrier()
    pltpu.sync_copy(acc_s.at[pl.ds(s * _ZR, _ZR)],
                    out_hbm.at[c, pl.ds(s * _ZR, _ZR)])


@functools.lru_cache(maxsize=None)
def _build_deg_kernel():
    mesh = plsc.VectorSubcoreMesh(core_axis_name="c", subcore_axis_name="s")
    return functools.partial(
        pl.kernel,
        mesh=mesh,
        out_type=jax.ShapeDtypeStruct((B, NDUM, HG), F32),
        scratch_types=[
            pltpu.VMEM((_DGK, CHUNK), jnp.int32),
            pltpu.VMEM((CHUNK, HG), F32),
            pltpu.VMEM_SHARED((NDUM, HG), F32),
            pltpu.SemaphoreType.DMA,
            pltpu.SemaphoreType.DMA,
        ],
    )(_deg_body)


def _deg_call(ones_chunk, dstp, zeros128):
    return _build_deg_kernel()(ones_chunk, dstp, zeros128)[:, :N]


def _agg_call(hflat, srcs2, dstp, zeros128):
    return _build_agg_kernel(HG)(hflat, srcs2, dstp, zeros128)[:, :N]


def kernel(dynamic_features, static_features, edge_index, W_ih0, W_hh0, b_ih0,
           b_hh0, W_ih1, W_hh1, b_ih1, b_hh1, Ws, bs, Wc, bc, Wsp, bsp,
           W1, b1, W2, b2, Wl, bl):
    x2d = dynamic_features.reshape(B * N, T * DD)
    xs = static_features.reshape(B * N, DS)
    sg = static_features[0]

    src = edge_index[0].astype(jnp.int32)
    dst = edge_index[1].astype(jnp.int32)
    pad = EPAD - E
    srcp = jnp.concatenate([src, jnp.zeros((pad,), jnp.int32)])
    dstp = jnp.concatenate([dst, jnp.full((pad,), N, jnp.int32)])
    srcs2 = jnp.stack([srcp, srcp + N]).reshape(B, EPAD // CHUNK, CHUNK)
    zeros128 = jnp.zeros((NDUM, HG), F32)
    ones_chunk = jnp.ones((CHUNK, HG), F32)

    b0 = (b_ih0 + b_hh0).reshape(1, 4 * HL)
    b1s = (b_ih1 + b_hh1).reshape(1, 4 * HL)
    comb = _encode(x2d, xs, W_ih0.T, W_hh0.T, b0, W_ih1.T, W_hh1.T, b1s,
                   Ws.T, bs.reshape(1, -1), Wc[:, :HL].T, Wc[:, HL:].T,
                   bc.reshape(1, -1))

    deg2 = _deg_call(ones_chunk, dstp, zeros128)

    h1s, dinv = _gcn_pre(deg2, comb.reshape(B, N, HL), sg, Wsp.T,
                         bsp.reshape(1, -1), W1[:, :HL].T, W1[:, HL:].T)

    agg1 = _agg_call(h1s.reshape(B * N, HG), srcs2, dstp, zeros128)

    h2s = _gcn_mid(agg1, h1s, dinv, b1.reshape(1, -1), W2.T)

    agg2 = _agg_call(h2s.reshape(B * N, HG), srcs2, dstp, zeros128)

    res = _gcn_fin(agg2, h2s, dinv, b2.reshape(1, -1), Wl.T, bl.reshape(1, 1))
    return res[:, :, 0]
